# baseline (speedup 1.0000x reference)
import functools

import jax
import jax.numpy as jnp
from jax import lax
from jax.experimental import pallas as pl
from jax.experimental.pallas import tpu as pltpu
from jax.experimental.pallas import tpu_sc as plsc

BATCH, NPIX, CIN, COUT, KS = 2, 196608, 16, 16, 9
Q = NPIX // 8
NC, NS, L = 2, 16, 16
NW = NC * NS
RPT = NPIX // NW
CH = 256
NCH = RPT // CH
G = CH * KS
GSLICE = 128
NG = G // GSLICE
QB = 4096


def _tc_body(x8_ref, w8_ref, b8_ref, z_ref):
    z_ref[0, ...] = (
        jnp.dot(x8_ref[...], w8_ref[...], preferred_element_type=jnp.float32)
        + b8_ref[...]
    )


def _make_z(x8, w8, b8):
    return pl.pallas_call(
        _tc_body,
        grid=(Q // QB, KS),
        in_specs=[
            pl.BlockSpec((QB, 128), lambda i, k: (i, 0)),
            pl.BlockSpec((128, 128), lambda i, k: (0, k)),
            pl.BlockSpec((1, 128), lambda i, k: (0, 0)),
        ],
        out_specs=pl.BlockSpec((1, QB, 128), lambda i, k: (k, i, 0)),
        out_shape=jax.ShapeDtypeStruct((KS, Q, 128), jnp.float32),
        compiler_params=pltpu.CompilerParams(
            dimension_semantics=("parallel", "arbitrary")
        ),
    )(x8, w8, b8)


def _sc_body_impl(
    rpt, pix0, z_hbm, neigh_hbm, out_hbm, idx_v, rows_v, acc_v, koff_v, sem0, sem1
):
    nch = rpt // CH
    wid = lax.axis_index("s") * NC + lax.axis_index("c")
    pbase = pix0 + wid * rpt
    obase = wid * rpt
    iota16 = lax.iota(jnp.int32, L)
    sems = (sem0, sem1)

    for v in range(KS):
        koff_v[pl.ds(v * L, L)] = lax.rem(v * L + iota16, KS) * NPIX
    kpat = [koff_v[pl.ds(v * L, L)] for v in range(KS)]

    def stage(c, buf):
        p0 = pbase + c * CH
        pltpu.sync_copy(neigh_hbm.at[pl.ds(p0 * KS, G)], idx_v.at[buf])

        def idx_body(g, _):
            for v in range(KS):
                sl = pl.ds(g * (KS * L) + v * L, L)
                idx_v[buf, sl] = idx_v[buf, sl] + kpat[v]
            return 0

        lax.fori_loop(0, G // (KS * L), idx_body, 0)
        for j in range(NG):
            pltpu.async_copy(
                z_hbm.at[idx_v.at[buf, pl.ds(j * GSLICE, GSLICE)]],
                rows_v.at[buf, pl.ds(j * GSLICE, GSLICE), :],
                sems[buf],
            )

    def finish(c, buf):
        pltpu.make_async_copy(
            z_hbm.at[pl.ds(0, G)], rows_v.at[buf], sems[buf]
        ).wait()

        def acc_body(g, _):
            for u in range(4):
                p = g * 4 + u
                s = rows_v[buf, p * KS, :]
                for k in range(1, KS):
                    s = s + rows_v[buf, p * KS + k, :]
                acc_v[p, :] = s
            return 0

        lax.fori_loop(0, CH // 4, acc_body, 0)
        pltpu.sync_copy(acc_v, out_hbm.at[pl.ds(obase + c * CH, CH)])

    def pair_body(h, _):
        c0 = 2 * h
        stage(c0 + 1, 1)
        finish(c0, 0)

        @pl.when(h + 1 < nch // 2)
        def _():
            stage(c0 + 2, 0)

        finish(c0 + 1, 1)
        return 0

    stage(0, 0)
    lax.fori_loop(0, nch // 2, pair_body, 0)


def _make_sc_gather_sum(nrows, pix0):
    return functools.partial(
        pl.kernel,
        out_type=jax.ShapeDtypeStruct((nrows, COUT), jnp.float32),
        mesh=plsc.VectorSubcoreMesh(core_axis_name="c", subcore_axis_name="s"),
        scratch_types=[
            pltpu.VMEM((2, G), jnp.int32),
            pltpu.VMEM((2, G, COUT), jnp.float32),
            pltpu.VMEM((CH, COUT), jnp.float32),
            pltpu.VMEM((KS * L,), jnp.int32),
            pltpu.SemaphoreType.DMA,
            pltpu.SemaphoreType.DMA,
        ],
        compiler_params=pltpu.CompilerParams(use_tc_tiling_on_sc=False),
    )(functools.partial(_sc_body_impl, nrows // NW, pix0))


_sc_full = _make_sc_gather_sum(NPIX, 0)
_sc_half_a = _make_sc_gather_sum(NPIX // 2, 0)
_sc_half_b = _make_sc_gather_sum(NPIX // 2, NPIX // 2)


def kernel(x, neighbours, w, b):
    w2 = jnp.transpose(w, (2, 1, 0)).reshape(CIN, KS, COUT)
    w8 = jnp.einsum("mp,cko->kmcpo", jnp.eye(8, dtype=jnp.float32), w2)
    w8 = w8.reshape(KS, 8 * CIN, 8 * COUT).transpose(1, 0, 2).reshape(
        128, KS * 128
    ).astype(jnp.bfloat16)
    b8 = jnp.tile(b / KS, (8,)).reshape(1, 128)
    nf = neighbours.reshape(NPIX * KS)
    xh = x.astype(jnp.bfloat16)
    zfs = []
    for bi in range(BATCH):
        x8 = xh[bi].reshape(Q, 128)
        zfs.append(_make_z(x8, w8, b8).reshape(KS * NPIX, COUT))
    y0 = _sc_full(zfs[0], nf)
    y1a = _sc_half_a(zfs[1], nf)
    y1b = _sc_half_b(zfs[1], nf)
    y = jnp.concatenate([y0, y1a, y1b], axis=0)
    return y.reshape(BATCH, NPIX, COUT)


# speedup vs baseline: 1.0045x; 1.0045x over previous
import functools

import jax
import jax.numpy as jnp
from jax import lax
from jax.experimental import pallas as pl
from jax.experimental.pallas import tpu as pltpu
from jax.experimental.pallas import tpu_sc as plsc

BATCH, NPIX, CIN, COUT, KS = 2, 196608, 16, 16, 9
Q = NPIX // 8
NC, NS, L = 2, 16, 16
NW = NC * NS
RPT = NPIX // NW
CH = 256
NCH = RPT // CH
G = CH * KS
GSLICE = 128
NG = G // GSLICE
QB = 4096


def _tc_body(x8_ref, w8_ref, b8_ref, z_ref):
    z_ref[0, ...] = (
        jnp.dot(x8_ref[...], w8_ref[...], preferred_element_type=jnp.float32)
        + b8_ref[...]
    )


def _make_z(x8, w8, b8):
    return pl.pallas_call(
        _tc_body,
        grid=(Q // QB, KS),
        in_specs=[
            pl.BlockSpec((QB, 128), lambda i, k: (i, 0)),
            pl.BlockSpec((128, 128), lambda i, k: (0, k)),
            pl.BlockSpec((1, 128), lambda i, k: (0, 0)),
        ],
        out_specs=pl.BlockSpec((1, QB, 128), lambda i, k: (k, i, 0)),
        out_shape=jax.ShapeDtypeStruct((KS, Q, 128), jnp.float32),
        compiler_params=pltpu.CompilerParams(
            dimension_semantics=("parallel", "arbitrary")
        ),
    )(x8, w8, b8)


def _sc_body_impl(
    rpt, pix0, z_hbm, neigh_hbm, out_hbm, idx_v, rows_v, acc_v, koff_v, sem0, sem1
):
    nch = rpt // CH
    wid = lax.axis_index("s") * NC + lax.axis_index("c")
    pbase = pix0 + wid * rpt
    obase = wid * rpt
    iota16 = lax.iota(jnp.int32, L)
    sems = (sem0, sem1)

    for v in range(KS):
        koff_v[pl.ds(v * L, L)] = lax.rem(v * L + iota16, KS) * NPIX
    kpat = [koff_v[pl.ds(v * L, L)] for v in range(KS)]

    def stage(c, buf):
        p0 = pbase + c * CH
        pltpu.sync_copy(neigh_hbm.at[pl.ds(p0 * KS, G)], idx_v.at[buf])

        def idx_body(g, _):
            for v in range(KS):
                sl = pl.ds(g * (KS * L) + v * L, L)
                idx_v[buf, sl] = idx_v[buf, sl] + kpat[v]
            return 0

        lax.fori_loop(0, G // (KS * L), idx_body, 0)
        for j in range(NG):
            pltpu.async_copy(
                z_hbm.at[idx_v.at[buf, pl.ds(j * GSLICE, GSLICE)]],
                rows_v.at[buf, pl.ds(j * GSLICE, GSLICE), :],
                sems[buf],
            )

    def finish(c, buf):
        pltpu.make_async_copy(
            z_hbm.at[pl.ds(0, G)], rows_v.at[buf], sems[buf]
        ).wait()

        def acc_body(g, _):
            for u in range(4):
                p = g * 4 + u
                s = rows_v[buf, p * KS, :]
                for k in range(1, KS):
                    s = s + rows_v[buf, p * KS + k, :]
                acc_v[p, :] = s
            return 0

        lax.fori_loop(0, CH // 4, acc_body, 0)
        pltpu.sync_copy(acc_v, out_hbm.at[pl.ds(obase + c * CH, CH)])

    def pair_body(h, _):
        c0 = 2 * h
        stage(c0 + 1, 1)
        finish(c0, 0)

        @pl.when(h + 1 < nch // 2)
        def _():
            stage(c0 + 2, 0)

        finish(c0 + 1, 1)
        return 0

    stage(0, 0)
    lax.fori_loop(0, nch // 2, pair_body, 0)


def _make_sc_gather_sum(nrows, pix0):
    return functools.partial(
        pl.kernel,
        out_type=jax.ShapeDtypeStruct((nrows, COUT), jnp.float32),
        mesh=plsc.VectorSubcoreMesh(core_axis_name="c", subcore_axis_name="s"),
        scratch_types=[
            pltpu.VMEM((2, G), jnp.int32),
            pltpu.VMEM((2, G, COUT), jnp.float32),
            pltpu.VMEM((CH, COUT), jnp.float32),
            pltpu.VMEM((KS * L,), jnp.int32),
            pltpu.SemaphoreType.DMA,
            pltpu.SemaphoreType.DMA,
        ],
        compiler_params=pltpu.CompilerParams(use_tc_tiling_on_sc=False),
    )(functools.partial(_sc_body_impl, nrows // NW, pix0))


_sc_full = _make_sc_gather_sum(NPIX, 0)


def kernel(x, neighbours, w, b):
    w2 = jnp.transpose(w, (2, 1, 0)).reshape(CIN, KS, COUT)
    w8 = jnp.einsum("mp,cko->kmcpo", jnp.eye(8, dtype=jnp.float32), w2)
    w8 = w8.reshape(KS, 8 * CIN, 8 * COUT).transpose(1, 0, 2).reshape(
        128, KS * 128
    ).astype(jnp.bfloat16)
    b8 = jnp.tile(b / KS, (8,)).reshape(1, 128)
    nf = neighbours.reshape(NPIX * KS)
    xh = x.astype(jnp.bfloat16)
    zfs = []
    for bi in range(BATCH):
        x8 = xh[bi].reshape(Q, 128)
        zfs.append(_make_z(x8, w8, b8).reshape(KS * NPIX, COUT))
    y0 = _sc_full(zfs[0], nf)
    y1 = _sc_full(zfs[1], nf)
    y = jnp.concatenate([y0, y1], axis=0)
    return y.reshape(BATCH, NPIX, COUT)


# speedup vs baseline: 1.0381x; 1.0335x over previous
import functools

import jax
import jax.numpy as jnp
from jax import lax
from jax.experimental import pallas as pl
from jax.experimental.pallas import tpu as pltpu
from jax.experimental.pallas import tpu_sc as plsc

BATCH, NPIX, CIN, COUT, KS = 2, 196608, 16, 16, 9
Q = NPIX // 8
NC, NS, L = 2, 16, 16
NW = NC * NS
RPT = NPIX // NW
CH = 256
NCH = RPT // CH
G = CH * KS
GSLICE = 128
NG = G // GSLICE
QB = 4096


def _tc_body(x8_ref, w8_ref, b8_ref, z_ref):
    z_ref[0, ...] = (
        jnp.dot(x8_ref[...], w8_ref[...], preferred_element_type=jnp.float32)
        + b8_ref[...]
    )


def _make_z(x8, w8, b8):
    return pl.pallas_call(
        _tc_body,
        grid=(Q // QB, KS),
        in_specs=[
            pl.BlockSpec((QB, 128), lambda i, k: (i, 0)),
            pl.BlockSpec((128, 128), lambda i, k: (0, k)),
            pl.BlockSpec((1, 128), lambda i, k: (0, 0)),
        ],
        out_specs=pl.BlockSpec((1, QB, 128), lambda i, k: (k, i, 0)),
        out_shape=jax.ShapeDtypeStruct((KS, Q, 128), jnp.float32),
        compiler_params=pltpu.CompilerParams(
            dimension_semantics=("parallel", "arbitrary")
        ),
    )(x8, w8, b8)


def _sc_body_impl(
    rpt, pix0, z_hbm, neigh_hbm, out_hbm, idx_v, rows_v, acc_v, koff_v, sem0, sem1
):
    nch = rpt // CH
    wid = lax.axis_index("s") * NC + lax.axis_index("c")
    pbase = pix0 + wid * rpt
    obase = wid * rpt
    iota16 = lax.iota(jnp.int32, L)
    sems = (sem0, sem1)

    for v in range(KS):
        koff_v[pl.ds(v * L, L)] = lax.rem(v * L + iota16, KS) * NPIX
    kpat = [koff_v[pl.ds(v * L, L)] for v in range(KS)]

    def stage(c, buf):
        p0 = pbase + c * CH
        pltpu.sync_copy(neigh_hbm.at[pl.ds(p0 * KS, G)], idx_v.at[buf])

        def idx_body(g, _):
            for v in range(KS):
                sl = pl.ds(g * (KS * L) + v * L, L)
                idx_v[buf, sl] = idx_v[buf, sl] + kpat[v]
            return 0

        lax.fori_loop(0, G // (KS * L), idx_body, 0)
        for j in range(NG):
            pltpu.async_copy(
                z_hbm.at[idx_v.at[buf, pl.ds(j * GSLICE, GSLICE)]],
                rows_v.at[buf, pl.ds(j * GSLICE, GSLICE), :],
                sems[buf],
            )

    def finish(c, buf):
        pltpu.make_async_copy(
            z_hbm.at[pl.ds(0, G)], rows_v.at[buf], sems[buf]
        ).wait()

        def acc_body(g, _):
            for u in range(4):
                p = g * 4 + u
                s = rows_v[buf, p * KS, :]
                for k in range(1, KS):
                    s = s + rows_v[buf, p * KS + k, :]
                acc_v[p, :] = s
            return 0

        lax.fori_loop(0, CH // 4, acc_body, 0)
        pltpu.sync_copy(acc_v, out_hbm.at[pl.ds(obase + c * CH, CH)])

    def pair_body(h, _):
        c0 = 2 * h
        stage(c0 + 1, 1)
        finish(c0, 0)

        @pl.when(h + 1 < nch // 2)
        def _():
            stage(c0 + 2, 0)

        finish(c0 + 1, 1)
        return 0

    stage(0, 0)
    lax.fori_loop(0, nch // 2, pair_body, 0)


def _make_sc_gather_sum(nrows, pix0):
    return functools.partial(
        pl.kernel,
        out_type=jax.ShapeDtypeStruct((nrows, COUT), jnp.float32),
        mesh=plsc.VectorSubcoreMesh(core_axis_name="c", subcore_axis_name="s"),
        scratch_types=[
            pltpu.VMEM((2, G), jnp.int32),
            pltpu.VMEM((2, G, COUT), jnp.float32),
            pltpu.VMEM((CH, COUT), jnp.float32),
            pltpu.VMEM((KS * L,), jnp.int32),
            pltpu.SemaphoreType.DMA,
            pltpu.SemaphoreType.DMA,
        ],
        compiler_params=pltpu.CompilerParams(use_tc_tiling_on_sc=False),
    )(functools.partial(_sc_body_impl, nrows // NW, pix0))


_sc_full = _make_sc_gather_sum(NPIX, 0)


def kernel(x, neighbours, w, b):
    w2 = jnp.transpose(w, (2, 1, 0)).reshape(CIN, KS, COUT)
    w8 = jnp.einsum("mp,cko->kmcpo", jnp.eye(8, dtype=jnp.float32), w2)
    w8 = w8.reshape(KS, 8 * CIN, 8 * COUT).transpose(1, 0, 2).reshape(
        128, KS * 128
    ).astype(jnp.bfloat16)
    b8 = jnp.tile(b / KS, (8,)).reshape(1, 128)
    nf = neighbours.reshape(NPIX * KS)
    xh = x.astype(jnp.bfloat16)
    zfs = []
    for bi in range(BATCH):
        x8 = xh[bi].reshape(Q, 128)
        zfs.append(_make_z(x8, w8, b8).reshape(KS * NPIX, COUT))
    y0 = _sc_full(zfs[0], nf)
    y1 = _sc_full(zfs[1], nf)
    return jnp.stack([y0, y1]).reshape(BATCH, NPIX, COUT)
